# baseline (speedup 1.0000x reference)
import functools

import jax
import jax.numpy as jnp
from jax import lax
from jax.experimental import pallas as pl
from jax.experimental.pallas import tpu as pltpu
from jax.experimental.pallas import tpu_sc as plsc

N_NODES = 10000
D_FEAT = 128
N_PAIRS = 320000

NC = 2
NS = 16
NW = NC * NS
P_W = N_PAIRS // NW
CHUNK = 80
NCHUNK = P_W // CHUNK
ROWS = 3 * CHUNK


def _make_sc_kernel():
    mesh = plsc.VectorSubcoreMesh(core_axis_name="c", subcore_axis_name="s")

    @functools.partial(
        pl.kernel,
        mesh=mesh,
        compiler_params=pltpu.CompilerParams(needs_layout_passes=False,
                                             use_tc_tiling_on_sc=False),
        out_type=jax.ShapeDtypeStruct((NW, 16), jnp.float32),
        scratch_types=[
            pltpu.VMEM((3 * P_W,), jnp.int32),
            pltpu.VMEM((ROWS, D_FEAT // 2), jnp.int32),
            pltpu.VMEM((ROWS, D_FEAT // 2), jnp.int32),
            pltpu.VMEM((16,), jnp.float32),
            pltpu.VMEM((16,), jnp.float32),
            pltpu.VMEM_SHARED((N_NODES, D_FEAT // 2), jnp.int32),
            pltpu.SemaphoreType.DMA,
            pltpu.SemaphoreType.DMA,
        ],
    )
    def sc_loss(tab_hbm, comb_hbm, gam_hbm, out_hbm,
                cidx, rbuf0, rbuf1, gv, resv, stab, sem0, sem1):
        cid = lax.axis_index("c")
        sid = lax.axis_index("s")
        wid = sid * NC + cid

        pltpu.sync_copy(comb_hbm.at[pl.ds(wid * 3 * P_W, 3 * P_W)], cidx)
        pltpu.sync_copy(gam_hbm, gv)

        rows_per_sub = N_NODES // NS
        pltpu.sync_copy(tab_hbm.at[pl.ds(sid * rows_per_sub, rows_per_sub)],
                        stab.at[pl.ds(sid * rows_per_sub, rows_per_sub)])
        plsc.subcore_barrier()
        g0 = gv[...][0]
        zero16 = jnp.zeros((16,), jnp.float32)

        bufs = ((rbuf0, sem0), (rbuf1, sem1))

        def start(b, ci):
            rb, sem = bufs[b]
            pltpu.async_copy(stab.at[cidx.at[pl.ds(ci * ROWS, ROWS)]],
                             rb, sem)

        def wait(b):
            rb, sem = bufs[b]
            pltpu.make_async_copy(tab_hbm.at[pl.ds(0, ROWS)], rb, sem).wait()

        def compute(b, loss):
            rb, _ = bufs[b]

            def pair_body(p, loss):
                acc_a = zero16
                acc_b = zero16
                for c in range(D_FEAT // 32):
                    sl = pl.ds(c * 16, 16)
                    l32 = plsc.bitcast(rb[p, sl], jnp.bfloat16)
                    d32 = (plsc.bitcast(rb[2 * CHUNK + p, sl], jnp.bfloat16)
                           - plsc.bitcast(rb[CHUNK + p, sl], jnp.bfloat16))
                    prod = l32 * d32
                    pa, pb = plsc.unpack(
                        prod, format=plsc.PackFormat.INTERLEAVED,
                        preferred_element_type=jnp.float32)
                    acc_a = acc_a + pa
                    acc_b = acc_b + pb
                m = jnp.sum(acc_a + acc_b) + g0
                return loss + jnp.maximum(m, 0.0)

            return plsc.parallel_loop(0, CHUNK, carry=loss,
                                      unroll=8)(pair_body)

        start(0, 0)

        def body(i, loss):
            c0 = 2 * i
            start(1, c0 + 1)
            wait(0)
            loss = compute(0, loss)
            start(0, c0 + 2)
            wait(1)
            return compute(1, loss)

        loss = lax.fori_loop(0, (NCHUNK - 1) // 2, body, jnp.float32(0.0))
        wait(0)
        loss = compute(0, loss)
        resv[...] = jnp.full((16,), loss, jnp.float32)
        pltpu.sync_copy(resv, out_hbm.at[wid])

    return sc_loss


_sc_loss = _make_sc_kernel()


def kernel(out, left, pos_right, neg_right, single_gamma):
    out = lax.bitcast_convert_type(
        out.astype(jnp.bfloat16).reshape(N_NODES, D_FEAT // 2, 2),
        jnp.int32)
    shaped = [x.astype(jnp.int32).reshape(NW, NCHUNK, CHUNK)
              for x in (left, pos_right, neg_right)]
    comb = jnp.stack(shaped, axis=2).reshape(-1)
    gam = jnp.full((16,), single_gamma, jnp.float32)
    partials = _sc_loss(out, comb, gam)
    return jnp.sum(partials[:, 0]) / left.shape[0]


# speedup vs baseline: 1.0022x; 1.0022x over previous
import functools

import jax
import jax.numpy as jnp
from jax import lax
from jax.experimental import pallas as pl
from jax.experimental.pallas import tpu as pltpu
from jax.experimental.pallas import tpu_sc as plsc

N_NODES = 10000
D_FEAT = 128
N_PAIRS = 320000

NC = 2
NS = 16
NW = NC * NS
P_W = N_PAIRS // NW
CHUNK = 80
NCHUNK = P_W // CHUNK
ROWS = 3 * CHUNK


def _make_sc_kernel():
    mesh = plsc.VectorSubcoreMesh(core_axis_name="c", subcore_axis_name="s")

    @functools.partial(
        pl.kernel,
        mesh=mesh,
        compiler_params=pltpu.CompilerParams(needs_layout_passes=False,
                                             use_tc_tiling_on_sc=False),
        out_type=jax.ShapeDtypeStruct((NW, 16), jnp.float32),
        scratch_types=[
            pltpu.VMEM((3 * P_W,), jnp.int32),
            pltpu.VMEM((ROWS, D_FEAT // 2), jnp.int32),
            pltpu.VMEM((ROWS, D_FEAT // 2), jnp.int32),
            pltpu.VMEM((16,), jnp.float32),
            pltpu.VMEM((16,), jnp.float32),
            pltpu.VMEM_SHARED((N_NODES, D_FEAT // 2), jnp.int32),
            pltpu.SemaphoreType.DMA,
            pltpu.SemaphoreType.DMA,
        ],
    )
    def sc_loss(tab_hbm, comb_hbm, gam_hbm, out_hbm,
                cidx, rbuf0, rbuf1, gv, resv, stab, sem0, sem1):
        cid = lax.axis_index("c")
        sid = lax.axis_index("s")
        wid = sid * NC + cid

        pltpu.sync_copy(comb_hbm.at[pl.ds(wid * 3 * P_W, 3 * P_W)], cidx)
        pltpu.sync_copy(gam_hbm, gv)

        rows_per_sub = N_NODES // NS
        pltpu.sync_copy(tab_hbm.at[pl.ds(sid * rows_per_sub, rows_per_sub)],
                        stab.at[pl.ds(sid * rows_per_sub, rows_per_sub)])
        plsc.subcore_barrier()
        g0 = gv[...][0]
        zero16 = jnp.zeros((16,), jnp.float32)

        bufs = ((rbuf0, sem0), (rbuf1, sem1))

        NSPLIT = 6
        SUB = ROWS // NSPLIT

        def start(b, ci):
            rb, sem = bufs[b]
            for k in range(NSPLIT):
                pltpu.async_copy(
                    stab.at[cidx.at[pl.ds(ci * ROWS + k * SUB, SUB)]],
                    rb.at[pl.ds(k * SUB, SUB)], sem)

        def wait(b):
            rb, sem = bufs[b]
            pltpu.make_async_copy(tab_hbm.at[pl.ds(0, ROWS)], rb, sem).wait()

        def compute(b, loss):
            rb, _ = bufs[b]

            def pair_body(p, loss):
                acc_a = zero16
                acc_b = zero16
                for c in range(D_FEAT // 32):
                    sl = pl.ds(c * 16, 16)
                    l32 = plsc.bitcast(rb[p, sl], jnp.bfloat16)
                    d32 = (plsc.bitcast(rb[2 * CHUNK + p, sl], jnp.bfloat16)
                           - plsc.bitcast(rb[CHUNK + p, sl], jnp.bfloat16))
                    prod = l32 * d32
                    pa, pb = plsc.unpack(
                        prod, format=plsc.PackFormat.INTERLEAVED,
                        preferred_element_type=jnp.float32)
                    acc_a = acc_a + pa
                    acc_b = acc_b + pb
                m = jnp.sum(acc_a + acc_b) + g0
                return loss + jnp.maximum(m, 0.0)

            return plsc.parallel_loop(0, CHUNK, carry=loss,
                                      unroll=8)(pair_body)

        start(0, 0)

        def body(i, loss):
            c0 = 2 * i
            start(1, c0 + 1)
            wait(0)
            loss = compute(0, loss)
            start(0, c0 + 2)
            wait(1)
            return compute(1, loss)

        loss = lax.fori_loop(0, (NCHUNK - 1) // 2, body, jnp.float32(0.0))
        wait(0)
        loss = compute(0, loss)
        resv[...] = jnp.full((16,), loss, jnp.float32)
        pltpu.sync_copy(resv, out_hbm.at[wid])

    return sc_loss


_sc_loss = _make_sc_kernel()


def kernel(out, left, pos_right, neg_right, single_gamma):
    out = lax.bitcast_convert_type(
        out.astype(jnp.bfloat16).reshape(N_NODES, D_FEAT // 2, 2),
        jnp.int32)
    shaped = [x.astype(jnp.int32).reshape(NW, NCHUNK, CHUNK)
              for x in (left, pos_right, neg_right)]
    comb = jnp.stack(shaped, axis=2).reshape(-1)
    gam = jnp.full((16,), single_gamma, jnp.float32)
    partials = _sc_loss(out, comb, gam)
    return jnp.sum(partials[:, 0]) / left.shape[0]


# speedup vs baseline: 1.2417x; 1.2390x over previous
import functools

import jax
import jax.numpy as jnp
from jax import lax
from jax.experimental import pallas as pl
from jax.experimental.pallas import tpu as pltpu
from jax.experimental.pallas import tpu_sc as plsc

N_NODES = 10000
D_FEAT = 128
N_PAIRS = 320000

NC = 2
NS = 16
NW = NC * NS
P_W = N_PAIRS // NW
CHUNK = 80
NCHUNK = P_W // CHUNK
NBUF = 3


def _make_sc_kernel():
    mesh = plsc.VectorSubcoreMesh(core_axis_name="c", subcore_axis_name="s")

    row_buf = pltpu.VMEM((CHUNK, D_FEAT // 2), jnp.int32)

    @functools.partial(
        pl.kernel,
        mesh=mesh,
        compiler_params=pltpu.CompilerParams(needs_layout_passes=False,
                                             use_tc_tiling_on_sc=False),
        out_type=jax.ShapeDtypeStruct((NW, 16), jnp.float32),
        scratch_types=(
            [pltpu.VMEM((P_W,), jnp.int32)] * 3
            + [row_buf] * (3 * NBUF)
            + [pltpu.VMEM((16,), jnp.float32)] * 2
            + [pltpu.VMEM_SHARED((N_NODES, D_FEAT // 2), jnp.int32)]
            + [pltpu.SemaphoreType.DMA] * NBUF
        ),
    )
    def sc_loss(tab_hbm, left_hbm, pos_hbm, neg_hbm, gam_hbm, out_hbm,
                lidx, pidx, nidx,
                lr0, pr0, nr0, lr1, pr1, nr1, lr2, pr2, nr2,
                gv, resv, stab, sem0, sem1, sem2):
        cid = lax.axis_index("c")
        sid = lax.axis_index("s")
        wid = sid * NC + cid
        base = wid * P_W

        pltpu.sync_copy(left_hbm.at[pl.ds(base, P_W)], lidx)
        pltpu.sync_copy(pos_hbm.at[pl.ds(base, P_W)], pidx)
        pltpu.sync_copy(neg_hbm.at[pl.ds(base, P_W)], nidx)
        pltpu.sync_copy(gam_hbm, gv)

        rows_per_sub = N_NODES // NS
        pltpu.sync_copy(tab_hbm.at[pl.ds(sid * rows_per_sub, rows_per_sub)],
                        stab.at[pl.ds(sid * rows_per_sub, rows_per_sub)])
        plsc.subcore_barrier()
        g0 = gv[...][0]
        zero16 = jnp.zeros((16,), jnp.float32)

        bufs = ((lr0, pr0, nr0, sem0), (lr1, pr1, nr1, sem1),
                (lr2, pr2, nr2, sem2))

        def start(b, ci):
            lr, pr, nr, sem = bufs[b]
            off = ci * CHUNK
            pltpu.async_copy(stab.at[lidx.at[pl.ds(off, CHUNK)]], lr, sem)
            pltpu.async_copy(stab.at[pidx.at[pl.ds(off, CHUNK)]], pr, sem)
            pltpu.async_copy(stab.at[nidx.at[pl.ds(off, CHUNK)]], nr, sem)

        def wait(b):
            lr, pr, nr, sem = bufs[b]
            for dst in (lr, pr, nr):
                pltpu.make_async_copy(tab_hbm.at[pl.ds(0, CHUNK)], dst,
                                      sem).wait()

        def compute(b, loss):
            lr, pr, nr, _ = bufs[b]

            def pair_body(p, loss):
                acc_a = zero16
                acc_b = zero16
                for c in range(D_FEAT // 32):
                    sl = pl.ds(c * 16, 16)
                    l32 = plsc.bitcast(lr[p, sl], jnp.bfloat16)
                    d32 = (plsc.bitcast(nr[p, sl], jnp.bfloat16)
                           - plsc.bitcast(pr[p, sl], jnp.bfloat16))
                    prod = l32 * d32
                    pa, pb = plsc.unpack(
                        prod, format=plsc.PackFormat.INTERLEAVED,
                        preferred_element_type=jnp.float32)
                    acc_a = acc_a + pa
                    acc_b = acc_b + pb
                m = jnp.sum(acc_a + acc_b) + g0
                return loss + jnp.maximum(m, 0.0)

            return plsc.parallel_loop(0, CHUNK, carry=loss,
                                      unroll=8)(pair_body)

        for b in range(NBUF):
            start(b, b)

        def body(i, loss):
            c0 = 3 * i
            for b in range(NBUF):
                wait(b)
                loss = compute(b, loss)

                @pl.when(c0 + b + 3 < NCHUNK)
                def _():
                    start(b, c0 + b + 3)

            return loss

        loss = lax.fori_loop(0, NCHUNK // 3, body, jnp.float32(0.0))
        for k in range(NCHUNK % 3):
            b = (NCHUNK - (NCHUNK % 3) + k) % 3
            wait(b)
            loss = compute(b, loss)
        resv[...] = jnp.full((16,), loss, jnp.float32)
        pltpu.sync_copy(resv, out_hbm.at[wid])

    return sc_loss


_sc_loss = _make_sc_kernel()


def kernel(out, left, pos_right, neg_right, single_gamma):
    out = lax.bitcast_convert_type(
        out.astype(jnp.bfloat16).reshape(N_NODES, D_FEAT // 2, 2),
        jnp.int32)
    left = left.astype(jnp.int32)
    pos_right = pos_right.astype(jnp.int32)
    neg_right = neg_right.astype(jnp.int32)
    gam = jnp.full((16,), single_gamma, jnp.float32)
    partials = _sc_loss(out, left, pos_right, neg_right, gam)
    return jnp.sum(partials[:, 0]) / left.shape[0]
